# baseline (speedup 1.0000x reference)
import jax
import jax.numpy as jnp
from jax.experimental import pallas as pl
from jax.experimental.pallas import tpu as pltpu

B = 1024
P = 32
N = 20
NP = N * P
D = 6144
N_PAD = 24
DBLK = 512


def _decode_kernel(lab_ref, lat_ref, comp_ref, means_ref, out_ref, e_ref, h_ref):
    @pl.when(pl.program_id(0) == 0)
    def _build():
        lab = lab_ref[:, :1]
        cls = jax.lax.broadcasted_iota(jnp.int32, (B, NP), 1) // P
        lat_t = jnp.concatenate([lat_ref[...]] * N, axis=1)
        e_ref[...] = jnp.where(cls == lab, lat_t, 0.0).astype(jnp.bfloat16)
        hcls = jax.lax.broadcasted_iota(jnp.int32, (B, N_PAD), 1)
        h_ref[...] = jnp.where(hcls == lab, 1.0, 0.0)

    out_ref[...] = (
        jnp.dot(
            e_ref[...],
            comp_ref[...].astype(jnp.bfloat16),
            preferred_element_type=jnp.float32,
        )
        + jnp.dot(h_ref[...], means_ref[...], preferred_element_type=jnp.float32)
    )


def kernel(lattent_codes, object_labels, means, components):
    comp2d = components.reshape(NP, D)
    labels_b = jnp.broadcast_to(
        object_labels.astype(jnp.int32)[:, None], (B, 128)
    )
    means_pad = jnp.pad(means, ((0, N_PAD - N), (0, 0)))

    return pl.pallas_call(
        _decode_kernel,
        grid=(D // DBLK,),
        in_specs=[
            pl.BlockSpec((B, 128), lambda i: (0, 0)),
            pl.BlockSpec((B, P), lambda i: (0, 0)),
            pl.BlockSpec((NP, DBLK), lambda i: (0, i)),
            pl.BlockSpec((N_PAD, DBLK), lambda i: (0, i)),
        ],
        out_specs=pl.BlockSpec((B, DBLK), lambda i: (0, i)),
        out_shape=jax.ShapeDtypeStruct((B, D), jnp.float32),
        scratch_shapes=[
            pltpu.VMEM((B, NP), jnp.bfloat16),
            pltpu.VMEM((B, N_PAD), jnp.float32),
        ],
    )(labels_b, lattent_codes, comp2d, means_pad)


# speedup vs baseline: 1.0434x; 1.0434x over previous
import jax
import jax.numpy as jnp
from jax.experimental import pallas as pl
from jax.experimental.pallas import tpu as pltpu

B = 1024
P = 32
N = 20
NP = N * P
D = 6144
N_PAD = 24
DBLK = 2048


def _decode_kernel(lab_ref, lat_ref, comp_ref, means_ref, out_ref, e_ref, h_ref):
    @pl.when(pl.program_id(0) == 0)
    def _build():
        lab = lab_ref[:, :1]
        cls = jax.lax.broadcasted_iota(jnp.int32, (B, NP), 1) // P
        lat_t = jnp.concatenate([lat_ref[...]] * N, axis=1)
        e_ref[...] = jnp.where(cls == lab, lat_t, 0.0).astype(jnp.bfloat16)
        hcls = jax.lax.broadcasted_iota(jnp.int32, (B, N_PAD), 1)
        h_ref[...] = jnp.where(hcls == lab, 1.0, 0.0)

    out_ref[...] = (
        jnp.dot(
            e_ref[...],
            comp_ref[...].astype(jnp.bfloat16),
            preferred_element_type=jnp.float32,
        )
        + jnp.dot(h_ref[...], means_ref[...], preferred_element_type=jnp.float32)
    )


def kernel(lattent_codes, object_labels, means, components):
    comp2d = components.reshape(NP, D)
    labels_b = jnp.broadcast_to(
        object_labels.astype(jnp.int32)[:, None], (B, 128)
    )
    means_pad = jnp.pad(means, ((0, N_PAD - N), (0, 0)))

    return pl.pallas_call(
        _decode_kernel,
        grid=(D // DBLK,),
        in_specs=[
            pl.BlockSpec((B, 128), lambda i: (0, 0)),
            pl.BlockSpec((B, P), lambda i: (0, 0)),
            pl.BlockSpec((NP, DBLK), lambda i: (0, i)),
            pl.BlockSpec((N_PAD, DBLK), lambda i: (0, i)),
        ],
        out_specs=pl.BlockSpec((B, DBLK), lambda i: (0, i)),
        out_shape=jax.ShapeDtypeStruct((B, D), jnp.float32),
        scratch_shapes=[
            pltpu.VMEM((B, NP), jnp.bfloat16),
            pltpu.VMEM((B, N_PAD), jnp.float32),
        ],
    )(labels_b, lattent_codes, comp2d, means_pad)


# speedup vs baseline: 1.0921x; 1.0467x over previous
import jax
import jax.numpy as jnp
from jax.experimental import pallas as pl
from jax.experimental.pallas import tpu as pltpu

B = 1024
P = 32
N = 20
NP = N * P
D = 6144
N_PAD = 24
DBLK = 1024


def _decode_kernel(lab_ref, lat_ref, comp_ref, means_ref, out_ref, e_ref, h_ref):
    @pl.when(pl.program_id(0) == 0)
    def _build():
        lab = lab_ref[:, :1]
        cls = jax.lax.broadcasted_iota(jnp.int32, (B, NP), 1) // P
        lat_t = jnp.concatenate([lat_ref[...]] * N, axis=1)
        e_ref[...] = jnp.where(cls == lab, lat_t, 0.0).astype(jnp.bfloat16)
        hcls = jax.lax.broadcasted_iota(jnp.int32, (B, N_PAD), 1)
        h_ref[...] = jnp.where(hcls == lab, 1.0, 0.0)

    out_ref[...] = (
        jnp.dot(
            e_ref[...],
            comp_ref[...].astype(jnp.bfloat16),
            preferred_element_type=jnp.float32,
        )
        + jnp.dot(h_ref[...], means_ref[...], preferred_element_type=jnp.float32)
    )


def kernel(lattent_codes, object_labels, means, components):
    comp2d = components.reshape(NP, D)
    labels_b = jnp.broadcast_to(
        object_labels.astype(jnp.int32)[:, None], (B, 128)
    )
    means_pad = jnp.pad(means, ((0, N_PAD - N), (0, 0)))

    return pl.pallas_call(
        _decode_kernel,
        grid=(D // DBLK,),
        in_specs=[
            pl.BlockSpec((B, 128), lambda i: (0, 0)),
            pl.BlockSpec((B, P), lambda i: (0, 0)),
            pl.BlockSpec((NP, DBLK), lambda i: (0, i)),
            pl.BlockSpec((N_PAD, DBLK), lambda i: (0, i)),
        ],
        out_specs=pl.BlockSpec((B, DBLK), lambda i: (0, i)),
        out_shape=jax.ShapeDtypeStruct((B, D), jnp.float32),
        scratch_shapes=[
            pltpu.VMEM((B, NP), jnp.bfloat16),
            pltpu.VMEM((B, N_PAD), jnp.float32),
        ],
    )(labels_b, lattent_codes, comp2d, means_pad)


# speedup vs baseline: 1.1127x; 1.0188x over previous
import jax
import jax.numpy as jnp
from jax.experimental import pallas as pl
from jax.experimental.pallas import tpu as pltpu

B = 1024
P = 32
N = 20
NP = N * P
D = 6144
N_PAD = 24
DBLK = 1024
NBLK = D // DBLK


def _decode_kernel(lab_ref, lat_ref, comp_hbm, means_ref, out_hbm,
                   e_ref, h_ref, comp_buf, out_buf, in_sem, out_sem):
    def comp_copy(i, slot):
        return pltpu.make_async_copy(
            comp_hbm.at[:, pl.ds(i * DBLK, DBLK)],
            comp_buf.at[slot],
            in_sem.at[slot],
        )

    def out_copy(i, slot):
        return pltpu.make_async_copy(
            out_buf.at[slot],
            out_hbm.at[:, pl.ds(i * DBLK, DBLK)],
            out_sem.at[slot],
        )

    comp_copy(0, 0).start()

    lab = lab_ref[:, :1]
    cls = jax.lax.broadcasted_iota(jnp.int32, (B, NP), 1) // P
    lat_t = jnp.concatenate([lat_ref[...]] * N, axis=1)
    e_ref[...] = jnp.where(cls == lab, lat_t, 0.0).astype(jnp.bfloat16)
    hcls = jax.lax.broadcasted_iota(jnp.int32, (B, N_PAD), 1)
    h_ref[...] = jnp.where(hcls == lab, 1.0, 0.0)

    for i in range(NBLK):
        slot = i % 2
        if i + 1 < NBLK:
            comp_copy(i + 1, 1 - slot).start()
        comp_copy(i, slot).wait()
        acc = (
            jnp.dot(
                e_ref[...],
                comp_buf[slot].astype(jnp.bfloat16),
                preferred_element_type=jnp.float32,
            )
            + jnp.dot(
                h_ref[...],
                means_ref[:, pl.ds(i * DBLK, DBLK)],
                preferred_element_type=jnp.float32,
            )
        )
        if i >= 2:
            out_copy(i - 2, slot).wait()
        out_buf[slot] = acc
        out_copy(i, slot).start()

    out_copy(NBLK - 2, (NBLK - 2) % 2).wait()
    out_copy(NBLK - 1, (NBLK - 1) % 2).wait()


def kernel(lattent_codes, object_labels, means, components):
    comp2d = components.reshape(NP, D)
    labels_2d = object_labels.astype(jnp.int32)[:, None]
    means_pad = jnp.pad(means, ((0, N_PAD - N), (0, 0)))

    return pl.pallas_call(
        _decode_kernel,
        grid=(1,),
        in_specs=[
            pl.BlockSpec((B, 1), lambda i: (0, 0)),
            pl.BlockSpec((B, P), lambda i: (0, 0)),
            pl.BlockSpec(memory_space=pl.ANY),
            pl.BlockSpec((N_PAD, D), lambda i: (0, 0)),
        ],
        out_specs=pl.BlockSpec(memory_space=pl.ANY),
        out_shape=jax.ShapeDtypeStruct((B, D), jnp.float32),
        scratch_shapes=[
            pltpu.VMEM((B, NP), jnp.bfloat16),
            pltpu.VMEM((B, N_PAD), jnp.float32),
            pltpu.VMEM((2, NP, DBLK), jnp.float32),
            pltpu.VMEM((2, B, DBLK), jnp.float32),
            pltpu.SemaphoreType.DMA((2,)),
            pltpu.SemaphoreType.DMA((2,)),
        ],
    )(labels_2d, lattent_codes, comp2d, means_pad)


# speedup vs baseline: 1.2338x; 1.1088x over previous
import jax
import jax.numpy as jnp
from jax.experimental import pallas as pl
from jax.experimental.pallas import tpu as pltpu

B = 1024
P = 32
N = 20
NP = N * P
D = 6144
N_PAD = 32
K = NP + N_PAD
DBLK = 1024
NBLK = D // DBLK
CCHUNK = 512


def _decode_kernel(lab_ref, lat_ref, comp_hbm, means_hbm, out_hbm,
                   e_ref, w_buf, out_buf, csem, msem, out_sem):
    def comp_copy(i, slot):
        return pltpu.make_async_copy(
            comp_hbm.at[:, pl.ds(i * DBLK, DBLK)],
            w_buf.at[slot, pl.ds(0, NP)],
            csem.at[slot],
        )

    def means_copy(i, slot):
        return pltpu.make_async_copy(
            means_hbm.at[:, pl.ds(i * DBLK, DBLK)],
            w_buf.at[slot, pl.ds(NP, N_PAD)],
            msem.at[slot],
        )

    def out_copy(i, slot):
        return pltpu.make_async_copy(
            out_buf.at[slot],
            out_hbm.at[:, pl.ds(i * DBLK, DBLK)],
            out_sem.at[slot],
        )

    comp_copy(0, 0).start()
    means_copy(0, 0).start()

    lab = lab_ref[:, :1]
    j = jax.lax.broadcasted_iota(jnp.int32, (B, K), 1)
    cls = jnp.where(j < NP, j // P, j - NP)
    lat_t = jnp.concatenate([lat_ref[...]] * (K // P), axis=1)
    val = jnp.where(j < NP, lat_t, 1.0)
    e_ref[...] = jnp.where(cls == lab, val, 0.0).astype(jnp.bfloat16)

    for i in range(NBLK):
        slot = i % 2
        if i + 1 < NBLK:
            comp_copy(i + 1, 1 - slot).start()
            means_copy(i + 1, 1 - slot).start()
        comp_copy(i, slot).wait()
        means_copy(i, slot).wait()
        if i >= 2:
            out_copy(i - 2, slot).wait()
        for c in range(DBLK // CCHUNK):
            out_buf[slot, :, pl.ds(c * CCHUNK, CCHUNK)] = jnp.dot(
                e_ref[...],
                w_buf[slot, :, pl.ds(c * CCHUNK, CCHUNK)].astype(jnp.bfloat16),
                preferred_element_type=jnp.float32,
            )
        out_copy(i, slot).start()

    out_copy(NBLK - 2, (NBLK - 2) % 2).wait()
    out_copy(NBLK - 1, (NBLK - 1) % 2).wait()


def kernel(lattent_codes, object_labels, means, components):
    comp2d = components.reshape(NP, D)
    labels_2d = object_labels.astype(jnp.int32)[:, None]
    means_pad = jnp.pad(means, ((0, N_PAD - N), (0, 0)))

    return pl.pallas_call(
        _decode_kernel,
        grid=(1,),
        in_specs=[
            pl.BlockSpec((B, 1), lambda i: (0, 0)),
            pl.BlockSpec((B, P), lambda i: (0, 0)),
            pl.BlockSpec(memory_space=pl.ANY),
            pl.BlockSpec(memory_space=pl.ANY),
        ],
        out_specs=pl.BlockSpec(memory_space=pl.ANY),
        out_shape=jax.ShapeDtypeStruct((B, D), jnp.float32),
        scratch_shapes=[
            pltpu.VMEM((B, K), jnp.bfloat16),
            pltpu.VMEM((2, K, DBLK), jnp.float32),
            pltpu.VMEM((2, B, DBLK), jnp.float32),
            pltpu.SemaphoreType.DMA((2,)),
            pltpu.SemaphoreType.DMA((2,)),
            pltpu.SemaphoreType.DMA((2,)),
        ],
    )(labels_2d, lattent_codes, comp2d, means_pad)
